# baseline (speedup 1.0000x reference)
import jax
import jax.numpy as jnp
from jax import lax
from jax.experimental import pallas as pl
from jax.experimental.pallas import tpu as pltpu
from jax.experimental.pallas import tpu_sc as plsc

_B, _T, _D = 4, 8192, 1024
_ROWS = _B * _T
_NW = 32
_PER_W = _ROWS // _NW
_CHR = 16
_NCHUNK = _PER_W // _CHR
_LANES = 16
_VPR = _D // _LANES


def _sc_body(x_hbm, pos_hbm, out_hbm, xb0, pb0, xb1, pb1, sem_in, sem_out):
    c = lax.axis_index("c")
    s = lax.axis_index("s")
    wid = s * 2 + c
    base = wid * _PER_W
    pos_base = lax.rem(wid, 8) * _PER_W

    def fetch(i, xb, pb):
        ro = base + i * _CHR
        po = pos_base + i * _CHR
        pltpu.make_async_copy(x_hbm.at[pl.ds(ro, _CHR), :], xb, sem_in).start()
        pltpu.make_async_copy(pos_hbm.at[pl.ds(po, _CHR), :], pb, sem_in).start()

    def wait_in(xb, pb):
        pltpu.make_async_copy(x_hbm.at[pl.ds(base, _CHR), :], xb, sem_in).wait()
        pltpu.make_async_copy(pos_hbm.at[pl.ds(base, _CHR), :], pb, sem_in).wait()

    def wait_out(xb):
        pltpu.make_async_copy(xb, out_hbm.at[pl.ds(base, _CHR), :], sem_out).wait()

    def compute(xb, pb):
        return
        @plsc.parallel_loop(0, _CHR * _VPR, step=1, unroll=8)
        def _vbody(j):
            r = lax.shift_right_logical(j, 6)
            col = pl.multiple_of(
                lax.shift_left(lax.bitwise_and(j, _VPR - 1), 4), _LANES
            )
            sl = pl.ds(col, _LANES)
            xb[r, sl] = xb[r, sl] + pb[r, sl]

    def step(i, xb, pb, xo, po):
        wait_in(xb, pb)
        @pl.when(i >= 1)
        def _():
            wait_out(xo)

        @pl.when(i + 1 < _NCHUNK)
        def _():
            fetch(i + 1, xo, po)
        compute(xb, pb)
        pltpu.make_async_copy(
            xb, out_hbm.at[pl.ds(base + i * _CHR, _CHR), :], sem_out
        ).start()

    fetch(0, xb0, pb0)

    def loop(k, _):
        i = k * 2
        step(i, xb0, pb0, xb1, pb1)
        step(i + 1, xb1, pb1, xb0, pb0)
        return 0

    lax.fori_loop(0, _NCHUNK // 2, loop, 0)
    wait_out(xb1)


def kernel(x, pos_table):
    B, T, D = x.shape
    xf = x.reshape(B * T, D)
    mesh = plsc.VectorSubcoreMesh(core_axis_name="c", subcore_axis_name="s")
    run = pl.kernel(
        _sc_body,
        out_type=jax.ShapeDtypeStruct((_ROWS, _D), jnp.float32),
        mesh=mesh,
        scratch_types=[
            pltpu.VMEM((_CHR, _D), jnp.float32),
            pltpu.VMEM((_CHR, _D), jnp.float32),
            pltpu.VMEM((_CHR, _D), jnp.float32),
            pltpu.VMEM((_CHR, _D), jnp.float32),
            pltpu.SemaphoreType.DMA,
            pltpu.SemaphoreType.DMA,
        ],
    )
    out = run(xf, pos_table[:T])
    return out.reshape(B, T, D)


# speedup vs baseline: 1.1244x; 1.1244x over previous
import jax
import jax.numpy as jnp
from jax import lax
from jax.experimental import pallas as pl
from jax.experimental.pallas import tpu as pltpu
from jax.experimental.pallas import tpu_sc as plsc

_B, _T, _D = 4, 8192, 1024
_ROWS = _B * _T
_NW = 32
_PER_W = _ROWS // _NW
_CHR = 8
_NCHUNK = _PER_W // _CHR
_NSLOT = 4
_LANES = 16
_VPR = _D // _LANES


def _sc_body(x_hbm, pos_hbm, out_hbm, *refs):
    xbs = refs[0:_NSLOT]
    pbs = refs[_NSLOT : 2 * _NSLOT]
    sin = refs[2 * _NSLOT : 3 * _NSLOT]
    sout = refs[3 * _NSLOT : 4 * _NSLOT]

    c = lax.axis_index("c")
    s = lax.axis_index("s")
    wid = s * 2 + c
    base = wid * _PER_W
    pos_base = lax.rem(wid, 8) * _PER_W

    def fetch(i, sl):
        ro = base + i * _CHR
        po = pos_base + i * _CHR
        pltpu.make_async_copy(x_hbm.at[pl.ds(ro, _CHR), :], xbs[sl], sin[sl]).start()
        pltpu.make_async_copy(pos_hbm.at[pl.ds(po, _CHR), :], pbs[sl], sin[sl]).start()

    def wait_in(sl):
        pltpu.make_async_copy(x_hbm.at[pl.ds(base, _CHR), :], xbs[sl], sin[sl]).wait()
        pltpu.make_async_copy(pos_hbm.at[pl.ds(base, _CHR), :], pbs[sl], sin[sl]).wait()

    def wait_out(sl):
        pltpu.make_async_copy(
            xbs[sl], out_hbm.at[pl.ds(base, _CHR), :], sout[sl]
        ).wait()

    def compute(sl):
        xb = xbs[sl]
        pb = pbs[sl]

        @plsc.parallel_loop(0, _CHR * _VPR, step=1, unroll=8)
        def _vbody(j):
            r = lax.shift_right_logical(j, 6)
            col = pl.multiple_of(
                lax.shift_left(lax.bitwise_and(j, _VPR - 1), 4), _LANES
            )
            csl = pl.ds(col, _LANES)
            xb[r, csl] = xb[r, csl] + pb[r, csl]

    def step(i, sl):
        tsl = (sl + _NSLOT - 1) % _NSLOT
        wait_in(sl)

        @pl.when(i >= 1)
        def _():
            wait_out(tsl)

        @pl.when(i + _NSLOT - 1 < _NCHUNK)
        def _():
            fetch(i + _NSLOT - 1, tsl)
        compute(sl)
        pltpu.make_async_copy(
            xbs[sl], out_hbm.at[pl.ds(base + i * _CHR, _CHR), :], sout[sl]
        ).start()

    for j in range(_NSLOT - 1):
        fetch(j, j)

    def loop(k, _):
        for sl in range(_NSLOT):
            step(k * _NSLOT + sl, sl)
        return 0

    lax.fori_loop(0, _NCHUNK // _NSLOT, loop, 0)
    wait_out(_NSLOT - 1)


def kernel(x, pos_table):
    B, T, D = x.shape
    xf = x.reshape(B * T, D)
    mesh = plsc.VectorSubcoreMesh(core_axis_name="c", subcore_axis_name="s")
    scratch = [pltpu.VMEM((_CHR, _D), jnp.float32) for _ in range(2 * _NSLOT)]
    scratch += [pltpu.SemaphoreType.DMA for _ in range(2 * _NSLOT)]
    run = pl.kernel(
        _sc_body,
        out_type=jax.ShapeDtypeStruct((_ROWS, _D), jnp.float32),
        mesh=mesh,
        scratch_types=scratch,
    )
    out = run(xf, pos_table[:T])
    return out.reshape(B, T, D)
